# Initial kernel scaffold; baseline (speedup 1.0000x reference)
import jax
import jax.numpy as jnp
from jax.experimental import pallas as pl


def kernel(inputs, word_table, pos_table):
    raise NotImplementedError("write your pallas kernel here")


# speedup vs baseline: 2.4599x; 2.4599x over previous
import jax
import jax.numpy as jnp
from jax import lax
from jax.experimental import pallas as pl
from jax.experimental.pallas import tpu as pltpu
from jax.experimental.pallas import tpu_sc as plsc

SEQ = 200
DIM = 64
NC = 2
NS = 16
NW = NC * NS
CHUNK = 128


def _emb_body(idx_hbm, word_hbm, pos_hbm, out_hbm, idx_v, pos_v, gbuf_v,
              obuf_v, gsem):
    w = idx_hbm.shape[0] // NW
    g_cnt = w // CHUNK
    wid = lax.axis_index("s") * NC + lax.axis_index("c")
    base = wid * w

    pltpu.sync_copy(idx_hbm.at[pl.ds(base, w)], idx_v)
    pltpu.sync_copy(pos_hbm, pos_v.at[pl.ds(0, SEQ)])
    pltpu.sync_copy(pos_hbm, pos_v.at[pl.ds(SEQ, SEQ)])

    @pl.loop(0, g_cnt)
    def chunk_loop(g):
        pltpu.async_copy(
            word_hbm.at[idx_v.at[pl.ds(g * CHUNK, CHUNK)]], gbuf_v, gsem
        ).wait()
        phase = lax.rem(g * CHUNK, SEQ)

        @pl.loop(0, CHUNK)
        def row_loop(r):
            p = phase + r
            for c in range(DIM // 16):
                sl = pl.ds(c * 16, 16)
                obuf_v[r, sl] = gbuf_v[r, sl] + pos_v[p, sl]

        pltpu.sync_copy(obuf_v, out_hbm.at[pl.ds(base + g * CHUNK, CHUNK)])


def kernel(inputs, word_table, pos_table):
    b, seq = inputs.shape
    total = b * seq
    idx_flat = inputs.reshape(total).astype(jnp.int32)
    word_pad = jnp.pad(word_table, ((0, 0), (0, 128 - DIM)))

    mesh = plsc.VectorSubcoreMesh(core_axis_name="c", subcore_axis_name="s")
    call = pl.kernel(
        _emb_body,
        out_type=jax.ShapeDtypeStruct((total, DIM), jnp.float32),
        mesh=mesh,
        scratch_types=[
            pltpu.VMEM((total // NW,), jnp.int32),
            pltpu.VMEM((2 * SEQ, DIM), jnp.float32),
            pltpu.VMEM((CHUNK, 128), jnp.float32),
            pltpu.VMEM((CHUNK, DIM), jnp.float32),
            pltpu.SemaphoreType.DMA,
        ],
    )
    out = call(idx_flat, word_pad, pos_table)
    return out.reshape(b, seq, DIM)


# speedup vs baseline: 3.8563x; 1.5677x over previous
import jax
import jax.numpy as jnp
from jax import lax
from jax.experimental import pallas as pl
from jax.experimental.pallas import tpu as pltpu
from jax.experimental.pallas import tpu_sc as plsc

SEQ = 200
DIM = 64
NC = 2
NS = 16
NW = NC * NS
CHUNK = 64
NBUF = 4


def _emb_body(idx_hbm, word_hbm, pos_hbm, out_hbm, idx_v, pos_v, *bufs):
    gbufs = bufs[0:NBUF]
    obufs = bufs[NBUF:2 * NBUF]
    gsems = bufs[2 * NBUF:3 * NBUF]
    wsems = bufs[3 * NBUF:4 * NBUF]

    w = idx_hbm.shape[0] // NW
    g_cnt = w // CHUNK
    outer = g_cnt // NBUF
    wid = lax.axis_index("s") * NC + lax.axis_index("c")
    base = wid * w

    pltpu.sync_copy(idx_hbm.at[pl.ds(base, w)], idx_v)
    pltpu.sync_copy(pos_hbm, pos_v)

    def issue_gather(g, b):
        pltpu.async_copy(
            word_hbm.at[idx_v.at[pl.ds(g * CHUNK, CHUNK)]], gbufs[b], gsems[b]
        )

    for b in range(NBUF):
        issue_gather(b, b)

    @pl.loop(0, outer)
    def outer_loop(gg):
        for b in range(NBUF):
            g = gg * NBUF + b
            pltpu.make_async_copy(
                word_hbm.at[idx_v.at[pl.ds(0, CHUNK)]], gbufs[b], gsems[b]
            ).wait()

            @pl.when(gg > 0)
            def _():
                pltpu.make_async_copy(
                    obufs[b], out_hbm.at[pl.ds(base, CHUNK)], wsems[b]
                ).wait()

            phase = lax.rem(g * CHUNK, SEQ)

            @pl.loop(0, CHUNK, unroll=4)
            def row_loop(r):
                p = (phase + r) * DIM
                for c in range(DIM // 16):
                    sl = pl.ds(c * 16, 16)
                    obufs[b][r, sl] = gbufs[b][r, sl] + pos_v[pl.ds(p + c * 16, 16)]

            pltpu.async_copy(
                obufs[b], out_hbm.at[pl.ds(base + g * CHUNK, CHUNK)], wsems[b]
            )

            @pl.when(gg + 1 < outer)
            def _():
                issue_gather(g + NBUF, b)

    for b in range(NBUF):
        pltpu.make_async_copy(
            obufs[b], out_hbm.at[pl.ds(base, CHUNK)], wsems[b]
        ).wait()


def kernel(inputs, word_table, pos_table):
    b, seq = inputs.shape
    total = b * seq
    idx_flat = inputs.reshape(total).astype(jnp.int32)
    word_pad = jnp.pad(word_table, ((0, 0), (0, 128 - DIM)))
    pos_flat = jnp.concatenate([pos_table, pos_table]).reshape(2 * SEQ * DIM)

    mesh = plsc.VectorSubcoreMesh(core_axis_name="c", subcore_axis_name="s")
    call = pl.kernel(
        _emb_body,
        out_type=jax.ShapeDtypeStruct((total, DIM), jnp.float32),
        mesh=mesh,
        scratch_types=[
            pltpu.VMEM((total // NW,), jnp.int32),
            pltpu.VMEM((2 * SEQ * DIM,), jnp.float32),
        ]
        + [pltpu.VMEM((CHUNK, 128), jnp.float32) for _ in range(NBUF)]
        + [pltpu.VMEM((CHUNK, DIM), jnp.float32) for _ in range(NBUF)]
        + [pltpu.SemaphoreType.DMA for _ in range(2 * NBUF)],
    )
    out = call(idx_flat, word_pad, pos_flat)
    return out.reshape(b, seq, DIM)
